# Initial kernel scaffold; baseline (speedup 1.0000x reference)
import jax
import jax.numpy as jnp
from jax.experimental import pallas as pl


def kernel(points_xyz, center_xyz, features):
    raise NotImplementedError("write your pallas kernel here")


# speedup vs baseline: 17.4853x; 17.4853x over previous
import functools

import jax
import jax.numpy as jnp
import numpy as np
from jax import lax
from jax.experimental import pallas as pl
from jax.experimental.pallas import tpu as pltpu
from jax.experimental.pallas import tpu_sc as plsc

_K = 32
_R2 = np.float32(0.2 * 0.2)

_B, _N, _G, _C = 4, 8192, 1024, 128
_GB, _NB = 256, 512
_NW16 = _N // 16
_GPW = 128
_LISTN = 320
_UNROLL = 16
_RROWS = 4
_GBK = _GPW * _K

_W = np.zeros((_NB // 16, _NB), np.float32)
for _l in range(_NB):
    _W[_l // 16, _l] = float(1 << (_l % 16))
_W.setflags(write=False)


def _maskw_body(ctr_ref, pts_ref, w_ref, out_ref):
    c = ctr_ref[0]
    p = pts_ref[0]
    cc = c * c
    c2 = cc[:, 0:1] + cc[:, 1:2] + cc[:, 2:3]
    pp = p * p
    p2 = pp[0:1, :] + pp[1:2, :] + pp[2:3, :]
    cp = lax.dot_general(c, p, (((1,), (0,)), ((), ())),
                         preferred_element_type=jnp.float32)
    d2 = (c2 + p2) - 2.0 * cp
    mf = jnp.where(d2 < _R2, jnp.float32(1.0), jnp.float32(0.0))
    w = lax.dot_general(w_ref[...], mf, (((1,), (1,)), ((), ())),
                        preferred_element_type=jnp.float32)
    out_ref[0] = w.astype(jnp.int32)


def _mask_words(ctr, pts3, w):
    return pl.pallas_call(
        _maskw_body,
        grid=(_B, _G // _GB, _N // _NB),
        in_specs=[
            pl.BlockSpec((1, _GB, 3), lambda b, g, n: (b, g, 0)),
            pl.BlockSpec((1, 3, _NB), lambda b, g, n: (b, 0, n)),
            pl.BlockSpec((_NB // 16, _NB), lambda b, g, n: (0, 0)),
        ],
        out_specs=pl.BlockSpec((1, _NB // 16, _GB), lambda b, g, n: (b, n, g)),
        out_shape=jax.ShapeDtypeStruct((_B, _NW16, _G), jnp.int32),
    )(ctr, pts3, w)


_MESH = plsc.VectorSubcoreMesh(core_axis_name="c", subcore_axis_name="s",
                               num_cores=2, num_subcores=16)
_SC_PARAMS = pltpu.CompilerParams(use_tc_tiling_on_sc=False, needs_layout_passes=False)


@functools.partial(
    pl.kernel,
    out_type=(jax.ShapeDtypeStruct((_B * _G * _K,), jnp.int32),
              jax.ShapeDtypeStruct((_B * 3 * _G * _K,), jnp.float32)),
    mesh=_MESH,
    scratch_types=[
        pltpu.VMEM((_NW16, _GPW), jnp.int32),
        pltpu.VMEM((_N,), jnp.float32),
        pltpu.VMEM((_N,), jnp.float32),
        pltpu.VMEM((_N,), jnp.float32),
        pltpu.VMEM((3, _GPW), jnp.float32),
        pltpu.VMEM((_LISTN,), jnp.int32),
        pltpu.VMEM((_GPW * _K,), jnp.int32),
        pltpu.VMEM((3 * _GPW * _K,), jnp.float32),
    ],
    compiler_params=_SC_PARAMS,
)
def _ball_sc(mw_hbm, pts_hbm, ctr_hbm, idx_hbm, gx_hbm,
             mw_v, px_v, py_v, pz_v, ctr_v, list_v, idxb_v, gxb_v):
    wid = lax.axis_index("s") * 2 + lax.axis_index("c")
    b = wid // 8
    g0 = (wid % 8) * _GPW
    pltpu.sync_copy(mw_hbm.at[pl.ds(b * _NW16, _NW16), pl.ds(g0, _GPW)], mw_v)
    pltpu.sync_copy(pts_hbm.at[pl.ds((b * 3 + 0) * _N, _N)], px_v)
    pltpu.sync_copy(pts_hbm.at[pl.ds((b * 3 + 1) * _N, _N)], py_v)
    pltpu.sync_copy(pts_hbm.at[pl.ds((b * 3 + 2) * _N, _N)], pz_v)
    pltpu.sync_copy(ctr_hbm.at[:, pl.ds(b * _G + g0, _GPW)], ctr_v)
    iota16 = lax.iota(jnp.int32, 16)
    zeros16 = jnp.zeros((16,), jnp.int32)

    def per_center(g, carry):
        gsplat = jnp.full((16,), g, jnp.int32)
        cxv = plsc.load_gather(ctr_v, [zeros16, gsplat])
        cyv = plsc.load_gather(ctr_v, [zeros16 + 1, gsplat])
        czv = plsc.load_gather(ctr_v, [zeros16 + 2, gsplat])

        def cond(st):
            wi, wpv = st
            return jnp.logical_and(wi < _NW16 // _UNROLL, jnp.max(wpv) < _K)

        def gbody(st):
            wi, wpv = st
            wrow = plsc.load_gather(mw_v, [wi * _UNROLL + iota16, gsplat])
            for u in range(_UNROLL):
                widx = wi * _UNROLL + u
                wvec = jnp.full((16,), wrow[u], jnp.int32)
                bits = lax.shift_right_logical(wvec, iota16) & 1
                m = bits == 1
                pos = (wpv + plsc.cumsum(bits)) - 1
                inds = widx * 16 + iota16
                plsc.store_scatter(list_v, [pos], inds, mask=m)
                wpv = wpv + plsc.all_reduce_population_count(m)
            return wi + 1, wpv

        st0 = (jnp.int32(0), zeros16)
        _, wpv = lax.while_loop(cond, gbody, st0)
        wp = jnp.max(wpv)
        wpb = jnp.full((16,), wp, jnp.int32)
        l0 = list_v[pl.ds(0, 16)]
        l1 = list_v[pl.ds(16, 16)]
        firv = jnp.where(wpb > 0,
                         plsc.load_gather(list_v, [zeros16]), zeros16)
        i0 = jnp.where(iota16 < wpb, l0, firv)
        i1 = jnp.where((iota16 + 16) < wpb, l1, firv)
        base = g * _K
        idxb_v[pl.ds(base, 16)] = i0
        idxb_v[pl.ds(base + 16, 16)] = i1
        for h, iv in ((0, i0), (1, i1)):
            off = base + h * 16
            gxb_v[pl.ds(0 * _GBK + off, 16)] = plsc.load_gather(px_v, [iv]) - cxv
            gxb_v[pl.ds(1 * _GBK + off, 16)] = plsc.load_gather(py_v, [iv]) - cyv
            gxb_v[pl.ds(2 * _GBK + off, 16)] = plsc.load_gather(pz_v, [iv]) - czv
        return carry

    lax.fori_loop(0, _GPW, per_center, 0)
    pltpu.sync_copy(idxb_v, idx_hbm.at[pl.ds(b * _G * _K + g0 * _K, _GBK)])
    for d in range(3):
        pltpu.sync_copy(gxb_v.at[pl.ds(d * _GBK, _GBK)],
                        gx_hbm.at[pl.ds((b * 3 + d) * _G * _K + g0 * _K,
                                        _GBK)])


@functools.partial(
    pl.kernel,
    out_type=jax.ShapeDtypeStruct((_B * _C * _G * _K,), jnp.float32),
    mesh=_MESH,
    scratch_types=[
        pltpu.VMEM((_G * _K,), jnp.int32),
        pltpu.VMEM((_RROWS, _N), jnp.float32),
        pltpu.VMEM((_RROWS * _GBK,), jnp.float32),
    ],
    compiler_params=_SC_PARAMS,
)
def _gather_sc(idx_hbm, feat_hbm, out_hbm, idxb_v, fbuf_v, ob_v):
    wid = lax.axis_index("s") * 2 + lax.axis_index("c")
    b = wid // 8
    c0 = (wid % 8) * (_C // 8)
    gk = _G * _K
    pltpu.sync_copy(idx_hbm.at[pl.ds(b * gk, gk)], idxb_v)
    rsplat = [jnp.full((16,), r, jnp.int32) for r in range(_RROWS)]

    def cc_body(cc, carry):
        crow = c0 + cc * _RROWS
        pltpu.sync_copy(feat_hbm.at[pl.ds(b * _C + crow, _RROWS), :], fbuf_v)

        def gb_body(gb, carry2):
            def iv_body(iv, carry3):
                idxv = idxb_v[pl.ds(gb * _GBK + iv * 16, 16)]
                for r in range(_RROWS):
                    v = plsc.load_gather(fbuf_v, [rsplat[r], idxv])
                    ob_v[pl.ds(r * _GBK + iv * 16, 16)] = v
                return carry3

            lax.fori_loop(0, _GBK // 16, iv_body, 0)
            for r in range(_RROWS):
                pltpu.sync_copy(ob_v.at[pl.ds(r * _GBK, _GBK)],
                                out_hbm.at[pl.ds((b * _C + crow + r) * gk
                                                 + gb * _GBK, _GBK)])
            return carry2

        lax.fori_loop(0, gk // _GBK, gb_body, 0)
        return carry

    lax.fori_loop(0, (_C // 8) // _RROWS, cc_body, 0)


def kernel(points_xyz, center_xyz, features):
    pts3 = jnp.transpose(points_xyz, (0, 2, 1))
    mw = _mask_words(center_xyz, pts3, jnp.asarray(_W))
    ctr_t = jnp.transpose(center_xyz, (2, 0, 1)).reshape(3, _B * _G)
    idx, gx = _ball_sc(mw.reshape(_B * _NW16, _G),
                       pts3.reshape(_B * 3 * _N), ctr_t)
    gf = _gather_sc(idx, features.reshape(_B * _C, _N))
    return (gx.reshape(_B, 3, _G, _K), gf.reshape(_B, _C, _G, _K))
